# baseline (speedup 1.0000x reference)
import functools

import jax
import jax.numpy as jnp
from jax import lax
from jax.experimental import pallas as pl
from jax.experimental.pallas import tpu as pltpu
from jax.experimental.pallas import tpu_sc as plsc

ROWS = 4096
COLS = 200
CPAD = 256
DIM = 4
NUM_EMB = 8

_info = plsc.get_sparse_core_info()
NC = _info.num_cores
NS = _info.num_subcores
NW = NC * NS
R_PER_W = ROWS // NW
R_CHUNK = 32
N_CHUNK = R_PER_W // R_CHUNK

_OFFS = [*range(0, COLS - 15, 16)]
if _OFFS[-1] != COLS - 16:
    _OFFS.append(COLS - 16)


def _make_lookup():
    mesh = plsc.VectorSubcoreMesh(core_axis_name="c", subcore_axis_name="s")

    @functools.partial(
        pl.kernel,
        mesh=mesh,
        compiler_params=pltpu.CompilerParams(
            needs_layout_passes=False,
            use_tc_tiling_on_sc=False,
        ),
        out_type=jax.ShapeDtypeStruct((ROWS, CPAD, DIM), jnp.float32),
        scratch_types=[
            pltpu.VMEM((DIM * NUM_EMB,), jnp.float32),
            pltpu.VMEM((R_CHUNK, COLS), jnp.int32),
            pltpu.VMEM((R_CHUNK, CPAD, DIM), jnp.float32),
        ],
    )
    def lookup(ids_hbm, table_hbm, out_hbm, table_v, idx_v, out_v):
        wid = lax.axis_index("s") * NC + lax.axis_index("c")
        r0 = wid * R_PER_W
        pltpu.sync_copy(table_hbm, table_v)

        lane = lax.iota(jnp.int32, 16)
        cols = [lane + off for off in _OFFS]
        dims = [jnp.full((16,), d, dtype=jnp.int32) for d in range(DIM)]

        for h in range(N_CHUNK):
            pltpu.sync_copy(ids_hbm.at[pl.ds(r0 + h * R_CHUNK, R_CHUNK)], idx_v)

            @plsc.parallel_loop(0, R_CHUNK, unroll=2)
            def body(i):
                dst = out_v.at[i]
                for c, off in enumerate(_OFFS):
                    ids16 = idx_v[i, pl.ds(off, 16)]
                    for d in range(DIM):
                        vals = plsc.load_gather(
                            table_v.at[pl.ds(NUM_EMB * d, NUM_EMB)], [ids16]
                        )
                        plsc.store_scatter(dst, [cols[c], dims[d]], vals)

            pltpu.sync_copy(out_v, out_hbm.at[pl.ds(r0 + h * R_CHUNK, R_CHUNK)])

    return lookup


_lookup = _make_lookup()


def kernel(input_ids, embed_table):
    out = _lookup(input_ids.astype(jnp.int32), embed_table.T.reshape(-1))
    return out[:, :COLS, :]


# speedup vs baseline: 23.2810x; 23.2810x over previous
import functools

import jax
import jax.numpy as jnp
from jax import lax
from jax.experimental import pallas as pl
from jax.experimental.pallas import tpu as pltpu
from jax.experimental.pallas import tpu_sc as plsc

ROWS = 4096
COLS = 200
DIM = 4
NUM_EMB = 8

_info = plsc.get_sparse_core_info()
NC = _info.num_cores
NS = _info.num_subcores
NW = NC * NS
IBLK = ROWS // NW


def _make_lookup():
    mesh = plsc.VectorSubcoreMesh(core_axis_name="c", subcore_axis_name="s")

    @functools.partial(
        pl.kernel,
        mesh=mesh,
        compiler_params=pltpu.CompilerParams(
            needs_layout_passes=False,
            use_tc_tiling_on_sc=False,
        ),
        out_type=jax.ShapeDtypeStruct((COLS, NW, DIM, IBLK), jnp.float32),
        scratch_types=[
            pltpu.VMEM((DIM * NUM_EMB,), jnp.float32),
            pltpu.VMEM((COLS, IBLK), jnp.int32),
            pltpu.VMEM((COLS, DIM, IBLK), jnp.float32),
        ],
    )
    def lookup(ids_hbm, table_hbm, out_hbm, table_v, idx_v, out_v):
        wid = lax.axis_index("s") * NC + lax.axis_index("c")
        pltpu.sync_copy(table_hbm, table_v)
        pltpu.sync_copy(ids_hbm.at[:, pl.ds(wid * IBLK, IBLK)], idx_v)

        @plsc.parallel_loop(0, COLS, unroll=2)
        def body(j):
            for k in range(IBLK // 16):
                ids16 = idx_v[j, pl.ds(k * 16, 16)]
                for d in range(DIM):
                    vals = plsc.load_gather(
                        table_v.at[pl.ds(NUM_EMB * d, NUM_EMB)], [ids16]
                    )
                    out_v[j, d, pl.ds(k * 16, 16)] = vals

        pltpu.sync_copy(out_v, out_hbm.at[:, wid])

    return lookup


_lookup = _make_lookup()


def kernel(input_ids, embed_table):
    ids_t = input_ids.astype(jnp.int32).T
    b = _lookup(ids_t, embed_table.T.reshape(-1))
    return b.transpose(1, 3, 0, 2).reshape(ROWS, COLS, DIM)
